# Initial kernel scaffold; baseline (speedup 1.0000x reference)
import jax
import jax.numpy as jnp
from jax.experimental import pallas as pl


def kernel(pos, edge_index0, precomp0, connection0, cluster, edge_index1, precomp1, connection1, params):
    raise NotImplementedError("write your pallas kernel here")


# speedup vs baseline: 4.5506x; 4.5506x over previous
import functools

import jax
import jax.numpy as jnp
from jax.experimental import pallas as pl
from jax.experimental.pallas import tpu as pltpu

_N_NODES = 10000
_N_COARSE = 2500
_R = 6
_M = 2
_BE = 1000

_INTERPRET = False


def _edge_body(Mi, C, xj_ref, conn_ref, pc_ref, msg_ref):
    MiC = Mi * C
    xj = xj_ref[...]
    conn = conn_ref[...]
    pc = pc_ref[...]
    cr = conn[:, 0:1]
    ci = conn[:, 1:2]
    rot = []
    for m in range(_M):
        mm = min(m, Mi - 1)
        Xr = xj[:, 0 * MiC + mm * C:0 * MiC + mm * C + C]
        Xi = xj[:, 1 * MiC + mm * C:1 * MiC + mm * C + C]
        if m == 1:
            rot.append((Xr * cr - Xi * ci, Xr * ci + Xi * cr))
        else:
            rot.append((Xr, Xi))
    for r in range(_R):
        for m in range(_M):
            pr = pc[:, 0 * _M * _R + m * _R + r:0 * _M * _R + m * _R + r + 1]
            pi = pc[:, 1 * _M * _R + m * _R + r:1 * _M * _R + m * _R + r + 1]
            Ar, Ai = rot[m]
            k = r * _M * C + m * C
            msg_ref[:, 0, k:k + C] = pr * Ar - pi * Ai
            msg_ref[:, 1, k:k + C] = pr * Ai + pi * Ar


@functools.partial(jax.jit, static_argnames=("Mi", "C"))
def _edge_msgs(xj, conn, pcf, Mi, C):
    E = xj.shape[0]
    grid = E // _BE
    RMC = _R * _M * C
    return pl.pallas_call(
        functools.partial(_edge_body, Mi, C),
        grid=(grid,),
        in_specs=[
            pl.BlockSpec((_BE, 2 * Mi * C), lambda i: (i, 0)),
            pl.BlockSpec((_BE, 2), lambda i: (i, 0)),
            pl.BlockSpec((_BE, 2 * _M * _R), lambda i: (i, 0)),
        ],
        out_specs=pl.BlockSpec((_BE, 2, RMC), lambda i: (i, 0, 0)),
        out_shape=jax.ShapeDtypeStruct((E, 2, RMC), jnp.float32),
        interpret=_INTERPRET,
    )(xj, conn, pcf)


def _prep_pc(precomp):
    E = precomp.shape[0]
    return jnp.transpose(precomp, (0, 3, 2, 1)).reshape(E, 2 * _M * _R)


def _conv(x, src, dst, pcf, conn, W, b, Nn):
    Mi, C = x.shape[1], x.shape[2]
    f = jnp.transpose(x, (0, 3, 1, 2)).reshape(Nn, 2 * Mi * C)
    xj = f[src]
    msg = _edge_msgs(xj, conn, pcf, Mi, C)
    yp = jax.ops.segment_sum(msg, dst, num_segments=Nn)
    y = jnp.transpose(yp.reshape(Nn, 2, _R, _M, C), (0, 2, 3, 4, 1))
    out = jnp.einsum('nrmcz,rmco->nmoz', y, W)
    if b is not None:
        out = out.at[:, 0, :, 0].add(b)
    return out


def _c_relu(x, b):
    mag = jnp.sqrt(jnp.sum(x * x, axis=-1, keepdims=True) + 1e-12)
    scale = jax.nn.relu(mag + b[None, None, :, None]) / mag
    return x * scale


def _resnet_block(x, src, dst, pcf, conn, p, Nn):
    h = _conv(x, src, dst, pcf, conn, p['W1'], p['b1'], Nn)
    h = _c_relu(h, p['nb1'])
    h = _conv(h, src, dst, pcf, conn, p['W2'], p['b2'], Nn)
    sc = x
    if sc.shape[1] != h.shape[1]:
        sc = jnp.broadcast_to(sc, (sc.shape[0], h.shape[1], sc.shape[2], sc.shape[3]))
    if 'Ws' in p:
        sc = jnp.einsum('nmcz,co->nmoz', sc, p['Ws'])
    return _c_relu(h + sc, p['nb2'])


def kernel(pos, edge_index0, precomp0, connection0, cluster, edge_index1,
           precomp1, connection1, params):
    src0, dst0 = edge_index0[0], edge_index0[1]
    src1, dst1 = edge_index1[0], edge_index1[1]
    pcf0 = _prep_pc(precomp0)
    pcf1 = _prep_pc(precomp1)

    x = jax.nn.relu(pos @ params['lin0_W'] + params['lin0_b'])
    x = jnp.stack([x, jnp.zeros_like(x)], axis=-1)[:, None, :, :]

    x = _resnet_block(x, src0, dst0, pcf0, connection0, params['rb11'], _N_NODES)
    x_prepool = _resnet_block(x, src0, dst0, pcf0, connection0, params['rb12'], _N_NODES)

    ones = jnp.ones((x_prepool.shape[0],), dtype=x_prepool.dtype)
    counts = jnp.maximum(
        jax.ops.segment_sum(ones, cluster, num_segments=_N_COARSE), 1.0)
    xp = jax.ops.segment_sum(x_prepool, cluster,
                             num_segments=_N_COARSE) / counts[:, None, None, None]

    xp = _resnet_block(xp, src1, dst1, pcf1, connection1, params['rb21'], _N_COARSE)
    xp = _resnet_block(xp, src1, dst1, pcf1, connection1, params['rb22'], _N_COARSE)
    xp = _resnet_block(xp, src1, dst1, pcf1, connection1, params['rb31'], _N_COARSE)
    xp = _resnet_block(xp, src1, dst1, pcf1, connection1, params['rb32'], _N_COARSE)

    x_un = xp[cluster]
    x = jnp.concatenate([x_un, x_prepool], axis=2)

    x = _resnet_block(x, src0, dst0, pcf0, connection0, params['rb41'], _N_NODES)
    x = _resnet_block(x, src0, dst0, pcf0, connection0, params['rb42'], _N_NODES)
    x = _conv(x, src0, dst0, pcf0, connection0, params['convf_W'], None, _N_NODES)

    mag = jnp.sqrt(jnp.sum(x * x, axis=-1) + 1e-12)
    logits = mag.sum(axis=1) + params['bias']
    return jax.nn.log_softmax(logits, axis=1)


# speedup vs baseline: 42.9113x; 9.4298x over previous
import functools

import jax
import jax.numpy as jnp
from jax import lax
from jax.experimental import pallas as pl
from jax.experimental.pallas import tpu as pltpu
from jax.experimental.pallas import tpu_sc as plsc

_N_NODES = 10000
_N_COARSE = 2500
_R = 6
_M = 2
_BE = 1024
_NB = 128
_DR = 400


def _pad_edges(E):
    q = 16 * _NB
    return -(-E // q) * q


def _pad_nodes(N):
    return -(-N // _DR) * _DR


def _edge_body(Mi, C, NCK, xj_ref, conn_ref, pc_ref, msg_ref):
    MiC = Mi * C
    MC = _M * C
    RMC = _R * MC
    xj = xj_ref[...]
    conn = conn_ref[...]
    pc = pc_ref[...]
    cr = conn[:, 0:1]
    ci = conn[:, 1:2]
    rot = []
    for m in range(_M):
        mm = min(m, Mi - 1)
        Xr = xj[:, 0 * MiC + mm * C:0 * MiC + mm * C + C]
        Xi = xj[:, 1 * MiC + mm * C:1 * MiC + mm * C + C]
        if m == 1:
            rot.append((Xr * cr - Xi * ci, Xr * ci + Xi * cr))
        else:
            rot.append((Xr, Xi))
    for z in range(2):
        for r in range(_R):
            for m in range(_M):
                pr = pc[:, 0 * _M * _R + m * _R + r:0 * _M * _R + m * _R + r + 1]
                pi = pc[:, 1 * _M * _R + m * _R + r:1 * _M * _R + m * _R + r + 1]
                Ar, Ai = rot[m]
                val = pr * Ar - pi * Ai if z == 0 else pr * Ai + pi * Ar
                flat = z * RMC + r * MC + m * C
                s = 0
                while s < C:
                    ck, off = (flat + s) // 128, (flat + s) % 128
                    w = min(128 - off, C - s)
                    msg_ref[ck, :, off:off + w] = val[:, s:s + w]
                    s += w


@functools.partial(jax.jit, static_argnames=("Mi", "C", "NCK"))
def _edge_msgs(xj, conn, pcf, Mi, C, NCK):
    E = xj.shape[0]
    grid = E // _BE
    return pl.pallas_call(
        functools.partial(_edge_body, Mi, C, NCK),
        grid=(grid,),
        in_specs=[
            pl.BlockSpec((_BE, 2 * Mi * C), lambda i: (i, 0)),
            pl.BlockSpec((_BE, 2), lambda i: (i, 0)),
            pl.BlockSpec((_BE, 2 * _M * _R), lambda i: (i, 0)),
        ],
        out_specs=pl.BlockSpec((NCK, _BE, 128), lambda i: (0, i, 0)),
        out_shape=jax.ShapeDtypeStruct((NCK, E, 128), jnp.float32),
    )(xj, conn, pcf)


@functools.lru_cache(maxsize=None)
def _make_sc_scatter(EPC, N, NCK):
    NG = EPC // _NB
    assert NG % 16 == 0
    GT = NG // 16
    NZ = N // _DR
    mesh = plsc.VectorSubcoreMesh(core_axis_name="c", subcore_axis_name="s")

    @functools.partial(
        pl.kernel,
        out_type=jax.ShapeDtypeStruct((NCK, N, 128), jnp.float32),
        mesh=mesh,
        scratch_types=[
            pltpu.VMEM((GT, _NB), jnp.int32),
            pltpu.VMEM((_NB, 128), jnp.float32),
            pltpu.VMEM_SHARED((N, 128), jnp.float32),
        ],
    )
    def k(msg_hbm, dstr_hbm, zeros_hbm, y_hbm, idx_v, mbuf, acc):
        zc = lax.axis_index("c")
        sid = lax.axis_index("s")
        pltpu.sync_copy(dstr_hbm.at[zc, sid], idx_v)
        for j in range(-(-NCK // 2)):
            ck = j * 2 + zc

            @pl.when(ck < NCK)
            def _():
                for jz in range(-(-NZ // 16)):
                    g = jz * 16 + sid

                    @pl.when(g < NZ)
                    def _():
                        pltpu.sync_copy(zeros_hbm, acc.at[pl.ds(g * _DR, _DR)])

            plsc.subcore_barrier()

            @pl.when(ck < NCK)
            def _():
                def body(i, _):
                    base = (sid * GT + i) * _NB
                    pltpu.sync_copy(msg_hbm.at[ck, pl.ds(base, _NB)], mbuf)
                    pltpu.sync_copy(mbuf, acc.at[idx_v.at[i]], add=True)
                    return 0

                lax.fori_loop(0, GT, body, 0)

            plsc.subcore_barrier()

            @pl.when(ck < NCK)
            def _():
                for jz in range(-(-NZ // 16)):
                    g = jz * 16 + sid

                    @pl.when(g < NZ)
                    def _():
                        pltpu.sync_copy(acc.at[pl.ds(g * _DR, _DR)],
                                        y_hbm.at[ck, pl.ds(g * _DR, _DR)])

            plsc.subcore_barrier()

    return k


def _sc_scatter(msg, dstr, N):
    NCK, EPC, _ = msg.shape
    zeros = jnp.zeros((_DR, 128), jnp.float32)
    return _make_sc_scatter(EPC, N, NCK)(msg, dstr, zeros)


def _chunking(C):
    return 3 * C // 16


def _prep_pc(precomp):
    E = precomp.shape[0]
    p = jnp.transpose(precomp, (0, 3, 2, 1)).reshape(E, 2 * _M * _R)
    return jnp.concatenate(
        [p, jnp.zeros((_pad_edges(E) - E, 2 * _M * _R), jnp.float32)])


def _prep_conn(conn):
    E = conn.shape[0]
    return jnp.concatenate(
        [conn, jnp.zeros((_pad_edges(E) - E, 2), jnp.float32)])


def _prep_src(src, Nn):
    E = src.shape[0]
    return jnp.concatenate(
        [src.astype(jnp.int32),
         jnp.full((_pad_edges(E) - E,), Nn, jnp.int32)])


def _prep_dstr(dst):
    E = dst.shape[0]
    EP = _pad_edges(E)
    d = jnp.concatenate(
        [dst.astype(jnp.int32), jnp.zeros((EP - E,), jnp.int32)])
    return jnp.broadcast_to(d.reshape(1, 16, EP // _NB // 16, _NB),
                            (2, 16, EP // _NB // 16, _NB))


def _conv(x, srcp, dstr, pcf, conn, W, b, Nn):
    Mi, C = x.shape[1], x.shape[2]
    NCK = _chunking(C)
    NP = _pad_nodes(Nn)
    f = jnp.transpose(x, (0, 3, 1, 2)).reshape(Nn, 2 * Mi * C)
    f = jnp.concatenate([f, jnp.zeros((1, 2 * Mi * C), jnp.float32)])
    xj = f[srcp]
    msg = _edge_msgs(xj, conn, pcf, Mi, C, NCK)
    y4 = _sc_scatter(msg, dstr, NP)[:, :Nn]
    y3 = jnp.transpose(y4, (1, 0, 2)).reshape(Nn, 2, _R, _M, C)
    y = jnp.transpose(y3, (0, 2, 3, 4, 1))
    out = jnp.einsum('nrmcz,rmco->nmoz', y, W)
    if b is not None:
        out = out.at[:, 0, :, 0].add(b)
    return out


def _c_relu(x, b):
    mag = jnp.sqrt(jnp.sum(x * x, axis=-1, keepdims=True) + 1e-12)
    scale = jax.nn.relu(mag + b[None, None, :, None]) / mag
    return x * scale


def _resnet_block(x, src, dstr, pcf, conn, p, Nn):
    h = _conv(x, src, dstr, pcf, conn, p['W1'], p['b1'], Nn)
    h = _c_relu(h, p['nb1'])
    h = _conv(h, src, dstr, pcf, conn, p['W2'], p['b2'], Nn)
    sc = x
    if sc.shape[1] != h.shape[1]:
        sc = jnp.broadcast_to(sc, (sc.shape[0], h.shape[1], sc.shape[2], sc.shape[3]))
    if 'Ws' in p:
        sc = jnp.einsum('nmcz,co->nmoz', sc, p['Ws'])
    return _c_relu(h + sc, p['nb2'])


def _pool(x_prepool, cluster):
    N = x_prepool.shape[0]
    D = x_prepool.shape[1] * x_prepool.shape[2] * x_prepool.shape[3]
    EP = 2 * _pad_edges(N // 2)
    NCP = _pad_nodes(_N_COARSE)
    pay = jnp.concatenate(
        [x_prepool.reshape(N, D),
         jnp.ones((N, 1), jnp.float32),
         jnp.zeros((N, 128 - D - 1), jnp.float32)], axis=1)
    pay = jnp.concatenate(
        [pay, jnp.zeros((EP - N, 128), jnp.float32)], axis=0)
    msg = pay.reshape(2, EP // 2, 128)
    clp = jnp.concatenate(
        [cluster.astype(jnp.int32), jnp.zeros((EP - N,), jnp.int32)])
    dstr = clp.reshape(2, 16, (EP // 2) // _NB // 16, _NB)
    y = _sc_scatter(msg, dstr, NCP)
    s = y[0, :_N_COARSE] + y[1, :_N_COARSE]
    counts = jnp.maximum(s[:, D], 1.0)
    xp = s[:, :D].reshape(_N_COARSE, _M, 16, 2) / counts[:, None, None, None]
    return xp


def kernel(pos, edge_index0, precomp0, connection0, cluster, edge_index1,
           precomp1, connection1, params):
    src0 = _prep_src(edge_index0[0], _N_NODES)
    src1 = _prep_src(edge_index1[0], _N_COARSE)
    pcf0 = _prep_pc(precomp0)
    pcf1 = _prep_pc(precomp1)
    conn0 = _prep_conn(connection0)
    conn1 = _prep_conn(connection1)
    dstr0 = _prep_dstr(edge_index0[1])
    dstr1 = _prep_dstr(edge_index1[1])

    x = jax.nn.relu(pos @ params['lin0_W'] + params['lin0_b'])
    x = jnp.stack([x, jnp.zeros_like(x)], axis=-1)[:, None, :, :]

    x = _resnet_block(x, src0, dstr0, pcf0, conn0, params['rb11'], _N_NODES)
    x_prepool = _resnet_block(x, src0, dstr0, pcf0, conn0, params['rb12'], _N_NODES)

    xp = _pool(x_prepool, cluster)

    xp = _resnet_block(xp, src1, dstr1, pcf1, conn1, params['rb21'], _N_COARSE)
    xp = _resnet_block(xp, src1, dstr1, pcf1, conn1, params['rb22'], _N_COARSE)
    xp = _resnet_block(xp, src1, dstr1, pcf1, conn1, params['rb31'], _N_COARSE)
    xp = _resnet_block(xp, src1, dstr1, pcf1, conn1, params['rb32'], _N_COARSE)

    x_un = xp[cluster]
    x = jnp.concatenate([x_un, x_prepool], axis=2)

    x = _resnet_block(x, src0, dstr0, pcf0, conn0, params['rb41'], _N_NODES)
    x = _resnet_block(x, src0, dstr0, pcf0, conn0, params['rb42'], _N_NODES)
    x = _conv(x, src0, dstr0, pcf0, conn0, params['convf_W'], None, _N_NODES)

    mag = jnp.sqrt(jnp.sum(x * x, axis=-1) + 1e-12)
    logits = mag.sum(axis=1) + params['bias']
    return jax.nn.log_softmax(logits, axis=1)


# speedup vs baseline: 45.6650x; 1.0642x over previous
import functools

import jax
import jax.numpy as jnp
from jax import lax
from jax.experimental import pallas as pl
from jax.experimental.pallas import tpu as pltpu
from jax.experimental.pallas import tpu_sc as plsc

_N_NODES = 10000
_N_COARSE = 2500
_R = 6
_M = 2
_BE = 1024
_NB = 128
_DR = 400


def _pad_edges(E):
    q = 32 * _NB
    return -(-E // q) * q


def _pad_nodes(N):
    return -(-N // _DR) * _DR


def _edge_body(Mi, C, NCK, xj_ref, conn_ref, pc_ref, msg_ref):
    MiC = Mi * C
    MC = _M * C
    RMC = _R * MC
    xj = xj_ref[...]
    conn = conn_ref[...]
    pc = pc_ref[...]
    cr = conn[:, 0:1]
    ci = conn[:, 1:2]
    rot = []
    for m in range(_M):
        mm = min(m, Mi - 1)
        Xr = xj[:, 0 * MiC + mm * C:0 * MiC + mm * C + C]
        Xi = xj[:, 1 * MiC + mm * C:1 * MiC + mm * C + C]
        if m == 1:
            rot.append((Xr * cr - Xi * ci, Xr * ci + Xi * cr))
        else:
            rot.append((Xr, Xi))
    for z in range(2):
        for r in range(_R):
            for m in range(_M):
                pr = pc[:, 0 * _M * _R + m * _R + r:0 * _M * _R + m * _R + r + 1]
                pi = pc[:, 1 * _M * _R + m * _R + r:1 * _M * _R + m * _R + r + 1]
                Ar, Ai = rot[m]
                val = pr * Ar - pi * Ai if z == 0 else pr * Ai + pi * Ar
                flat = z * RMC + r * MC + m * C
                s = 0
                while s < C:
                    ck, off = (flat + s) // 128, (flat + s) % 128
                    w = min(128 - off, C - s)
                    msg_ref[ck, :, off:off + w] = val[:, s:s + w]
                    s += w


@functools.partial(jax.jit, static_argnames=("Mi", "C", "NCK"))
def _edge_msgs(xj, conn, pcf, Mi, C, NCK):
    E = xj.shape[0]
    grid = E // _BE
    return pl.pallas_call(
        functools.partial(_edge_body, Mi, C, NCK),
        grid=(grid,),
        in_specs=[
            pl.BlockSpec((_BE, 2 * Mi * C), lambda i: (i, 0)),
            pl.BlockSpec((_BE, 2), lambda i: (i, 0)),
            pl.BlockSpec((_BE, 2 * _M * _R), lambda i: (i, 0)),
        ],
        out_specs=pl.BlockSpec((NCK, _BE, 128), lambda i: (0, i, 0)),
        out_shape=jax.ShapeDtypeStruct((NCK, E, 128), jnp.float32),
    )(xj, conn, pcf)


@functools.lru_cache(maxsize=None)
def _make_sc_scatter(EPC, N, NCK):
    NG = EPC // _NB
    assert NG % 16 == 0
    GT = NG // 16
    NZ = N // _DR
    mesh = plsc.VectorSubcoreMesh(core_axis_name="c", subcore_axis_name="s")

    @functools.partial(
        pl.kernel,
        out_type=jax.ShapeDtypeStruct((NCK, N, 128), jnp.float32),
        mesh=mesh,
        scratch_types=[
            pltpu.VMEM((GT, _NB), jnp.int32),
            pltpu.VMEM((_NB, 128), jnp.float32),
            pltpu.VMEM((_NB, 128), jnp.float32),
            pltpu.VMEM_SHARED((N, 128), jnp.float32),
            pltpu.SemaphoreType.DMA,
            pltpu.SemaphoreType.DMA,
        ],
    )
    def k(msg_hbm, dstr_hbm, zeros_hbm, y_hbm, idx_v, mb0, mb1, acc, sm0, sm1):
        zc = lax.axis_index("c")
        sid = lax.axis_index("s")
        mbufs, sems = (mb0, mb1), (sm0, sm1)
        pltpu.sync_copy(dstr_hbm.at[zc, sid], idx_v)
        for j in range(-(-NCK // 2)):
            ck = j * 2 + zc

            @pl.when(ck < NCK)
            def _():
                for jz in range(-(-NZ // 16)):
                    g = jz * 16 + sid

                    @pl.when(g < NZ)
                    def _():
                        pltpu.sync_copy(zeros_hbm, acc.at[pl.ds(g * _DR, _DR)])

            plsc.subcore_barrier()

            @pl.when(ck < NCK)
            def _():
                def row(i):
                    return msg_hbm.at[ck, pl.ds((sid * GT + i) * _NB, _NB)]

                for b in range(2):
                    pltpu.make_async_copy(row(b), mbufs[b], sems[b]).start()

                def body(jj, _):
                    for b in range(2):
                        i = 2 * jj + b
                        pltpu.make_async_copy(row(i), mbufs[b], sems[b]).wait()
                        pltpu.sync_copy(mbufs[b], acc.at[idx_v.at[i]], add=True)

                        @pl.when(i + 2 < GT)
                        def _():
                            pltpu.make_async_copy(
                                row(i + 2), mbufs[b], sems[b]).start()

                    return 0

                lax.fori_loop(0, GT // 2, body, 0)

            plsc.subcore_barrier()

            @pl.when(ck < NCK)
            def _():
                for jz in range(-(-NZ // 16)):
                    g = jz * 16 + sid

                    @pl.when(g < NZ)
                    def _():
                        pltpu.sync_copy(acc.at[pl.ds(g * _DR, _DR)],
                                        y_hbm.at[ck, pl.ds(g * _DR, _DR)])

            plsc.subcore_barrier()

    return k


def _sc_scatter(msg, dstr, N):
    NCK, EPC, _ = msg.shape
    zeros = jnp.zeros((_DR, 128), jnp.float32)
    return _make_sc_scatter(EPC, N, NCK)(msg, dstr, zeros)


def _chunking(C):
    return 3 * C // 16


def _prep_pc(precomp):
    E = precomp.shape[0]
    p = jnp.transpose(precomp, (0, 3, 2, 1)).reshape(E, 2 * _M * _R)
    return jnp.concatenate(
        [p, jnp.zeros((_pad_edges(E) - E, 2 * _M * _R), jnp.float32)])


def _prep_conn(conn):
    E = conn.shape[0]
    return jnp.concatenate(
        [conn, jnp.zeros((_pad_edges(E) - E, 2), jnp.float32)])


def _prep_src(src, Nn):
    E = src.shape[0]
    return jnp.concatenate(
        [src.astype(jnp.int32),
         jnp.full((_pad_edges(E) - E,), Nn, jnp.int32)])


def _prep_dstr(dst):
    E = dst.shape[0]
    EP = _pad_edges(E)
    d = jnp.concatenate(
        [dst.astype(jnp.int32), jnp.zeros((EP - E,), jnp.int32)])
    return jnp.broadcast_to(d.reshape(1, 16, EP // _NB // 16, _NB),
                            (2, 16, EP // _NB // 16, _NB))


def _conv(x, srcp, dstr, pcf, conn, W, b, Nn):
    Mi, C = x.shape[1], x.shape[2]
    NCK = _chunking(C)
    NP = _pad_nodes(Nn)
    f = jnp.transpose(x, (0, 3, 1, 2)).reshape(Nn, 2 * Mi * C)
    f = jnp.concatenate([f, jnp.zeros((1, 2 * Mi * C), jnp.float32)])
    xj = f[srcp]
    msg = _edge_msgs(xj, conn, pcf, Mi, C, NCK)
    y4 = _sc_scatter(msg, dstr, NP)[:, :Nn]
    y3 = jnp.transpose(y4, (1, 0, 2)).reshape(Nn, 2, _R, _M, C)
    y = jnp.transpose(y3, (0, 2, 3, 4, 1))
    out = jnp.einsum('nrmcz,rmco->nmoz', y, W)
    if b is not None:
        out = out.at[:, 0, :, 0].add(b)
    return out


def _c_relu(x, b):
    mag = jnp.sqrt(jnp.sum(x * x, axis=-1, keepdims=True) + 1e-12)
    scale = jax.nn.relu(mag + b[None, None, :, None]) / mag
    return x * scale


def _resnet_block(x, src, dstr, pcf, conn, p, Nn):
    h = _conv(x, src, dstr, pcf, conn, p['W1'], p['b1'], Nn)
    h = _c_relu(h, p['nb1'])
    h = _conv(h, src, dstr, pcf, conn, p['W2'], p['b2'], Nn)
    sc = x
    if sc.shape[1] != h.shape[1]:
        sc = jnp.broadcast_to(sc, (sc.shape[0], h.shape[1], sc.shape[2], sc.shape[3]))
    if 'Ws' in p:
        sc = jnp.einsum('nmcz,co->nmoz', sc, p['Ws'])
    return _c_relu(h + sc, p['nb2'])


def _pool(x_prepool, cluster):
    N = x_prepool.shape[0]
    D = x_prepool.shape[1] * x_prepool.shape[2] * x_prepool.shape[3]
    EP = 2 * _pad_edges(N // 2)
    NCP = _pad_nodes(_N_COARSE)
    pay = jnp.concatenate(
        [x_prepool.reshape(N, D),
         jnp.ones((N, 1), jnp.float32),
         jnp.zeros((N, 128 - D - 1), jnp.float32)], axis=1)
    pay = jnp.concatenate(
        [pay, jnp.zeros((EP - N, 128), jnp.float32)], axis=0)
    msg = pay.reshape(2, EP // 2, 128)
    clp = jnp.concatenate(
        [cluster.astype(jnp.int32), jnp.zeros((EP - N,), jnp.int32)])
    dstr = clp.reshape(2, 16, (EP // 2) // _NB // 16, _NB)
    y = _sc_scatter(msg, dstr, NCP)
    s = y[0, :_N_COARSE] + y[1, :_N_COARSE]
    counts = jnp.maximum(s[:, D], 1.0)
    xp = s[:, :D].reshape(_N_COARSE, _M, 16, 2) / counts[:, None, None, None]
    return xp


def kernel(pos, edge_index0, precomp0, connection0, cluster, edge_index1,
           precomp1, connection1, params):
    src0 = _prep_src(edge_index0[0], _N_NODES)
    src1 = _prep_src(edge_index1[0], _N_COARSE)
    pcf0 = _prep_pc(precomp0)
    pcf1 = _prep_pc(precomp1)
    conn0 = _prep_conn(connection0)
    conn1 = _prep_conn(connection1)
    dstr0 = _prep_dstr(edge_index0[1])
    dstr1 = _prep_dstr(edge_index1[1])

    x = jax.nn.relu(pos @ params['lin0_W'] + params['lin0_b'])
    x = jnp.stack([x, jnp.zeros_like(x)], axis=-1)[:, None, :, :]

    x = _resnet_block(x, src0, dstr0, pcf0, conn0, params['rb11'], _N_NODES)
    x_prepool = _resnet_block(x, src0, dstr0, pcf0, conn0, params['rb12'], _N_NODES)

    xp = _pool(x_prepool, cluster)

    xp = _resnet_block(xp, src1, dstr1, pcf1, conn1, params['rb21'], _N_COARSE)
    xp = _resnet_block(xp, src1, dstr1, pcf1, conn1, params['rb22'], _N_COARSE)
    xp = _resnet_block(xp, src1, dstr1, pcf1, conn1, params['rb31'], _N_COARSE)
    xp = _resnet_block(xp, src1, dstr1, pcf1, conn1, params['rb32'], _N_COARSE)

    x_un = xp[cluster]
    x = jnp.concatenate([x_un, x_prepool], axis=2)

    x = _resnet_block(x, src0, dstr0, pcf0, conn0, params['rb41'], _N_NODES)
    x = _resnet_block(x, src0, dstr0, pcf0, conn0, params['rb42'], _N_NODES)
    x = _conv(x, src0, dstr0, pcf0, conn0, params['convf_W'], None, _N_NODES)

    mag = jnp.sqrt(jnp.sum(x * x, axis=-1) + 1e-12)
    logits = mag.sum(axis=1) + params['bias']
    return jax.nn.log_softmax(logits, axis=1)


# speedup vs baseline: 53.1866x; 1.1647x over previous
import functools

import jax
import jax.numpy as jnp
from jax import lax
from jax.experimental import pallas as pl
from jax.experimental.pallas import tpu as pltpu
from jax.experimental.pallas import tpu_sc as plsc

_N_NODES = 10000
_N_COARSE = 2500
_R = 6
_M = 2
_BE = 1024
_NB = 128
_DR = 400


def _pad_edges(E):
    q = 32 * _NB
    return -(-E // q) * q


def _pad_nodes(N):
    return -(-N // _DR) * _DR


def _edge_body(Mi, C, NCK, xj_ref, conn_ref, pc_ref, msg_ref):
    MiC = Mi * C
    MC = _M * C
    RMC = _R * MC
    xj = xj_ref[...]
    conn = conn_ref[...]
    pc = pc_ref[...]
    cr = conn[:, 0:1]
    ci = conn[:, 1:2]
    rot = []
    for m in range(_M):
        mm = min(m, Mi - 1)
        Xr = xj[:, 0 * MiC + mm * C:0 * MiC + mm * C + C]
        Xi = xj[:, 1 * MiC + mm * C:1 * MiC + mm * C + C]
        if m == 1:
            rot.append((Xr * cr - Xi * ci, Xr * ci + Xi * cr))
        else:
            rot.append((Xr, Xi))
    for z in range(2):
        for r in range(_R):
            for m in range(_M):
                pr = pc[:, 0 * _M * _R + m * _R + r:0 * _M * _R + m * _R + r + 1]
                pi = pc[:, 1 * _M * _R + m * _R + r:1 * _M * _R + m * _R + r + 1]
                Ar, Ai = rot[m]
                val = pr * Ar - pi * Ai if z == 0 else pr * Ai + pi * Ar
                flat = z * RMC + r * MC + m * C
                s = 0
                while s < C:
                    ck, off = (flat + s) // 128, (flat + s) % 128
                    w = min(128 - off, C - s)
                    msg_ref[ck, :, off:off + w] = val[:, s:s + w]
                    s += w


@functools.partial(jax.jit, static_argnames=("Mi", "C", "NCK"))
def _edge_msgs(xj, conn, pcf, Mi, C, NCK):
    E, FP = xj.shape
    grid = E // _BE
    return pl.pallas_call(
        functools.partial(_edge_body, Mi, C, NCK),
        grid=(grid,),
        in_specs=[
            pl.BlockSpec((_BE, FP), lambda i: (i, 0)),
            pl.BlockSpec((_BE, 2), lambda i: (i, 0)),
            pl.BlockSpec((_BE, 2 * _M * _R), lambda i: (i, 0)),
        ],
        out_specs=pl.BlockSpec((NCK, _BE, 128), lambda i: (0, i, 0)),
        out_shape=jax.ShapeDtypeStruct((NCK, E, 128), jnp.float32),
    )(xj, conn, pcf)


@functools.lru_cache(maxsize=None)
def _make_sc_scatter(EPC, N, NCK):
    NG = EPC // _NB
    assert NG % 16 == 0
    GT = NG // 16
    NZ = N // _DR
    mesh = plsc.VectorSubcoreMesh(core_axis_name="c", subcore_axis_name="s")

    @functools.partial(
        pl.kernel,
        out_type=jax.ShapeDtypeStruct((NCK, N, 128), jnp.float32),
        mesh=mesh,
        scratch_types=[
            pltpu.VMEM((GT, _NB), jnp.int32),
            pltpu.VMEM((_NB, 128), jnp.float32),
            pltpu.VMEM((_NB, 128), jnp.float32),
            pltpu.VMEM_SHARED((N, 128), jnp.float32),
            pltpu.SemaphoreType.DMA,
            pltpu.SemaphoreType.DMA,
        ],
    )
    def k(msg_hbm, dstr_hbm, zeros_hbm, y_hbm, idx_v, mb0, mb1, acc, sm0, sm1):
        zc = lax.axis_index("c")
        sid = lax.axis_index("s")
        mbufs, sems = (mb0, mb1), (sm0, sm1)
        pltpu.sync_copy(dstr_hbm.at[zc, sid], idx_v)
        for j in range(-(-NCK // 2)):
            ck = j * 2 + zc

            @pl.when(ck < NCK)
            def _():
                for jz in range(-(-NZ // 16)):
                    g = jz * 16 + sid

                    @pl.when(g < NZ)
                    def _():
                        pltpu.sync_copy(zeros_hbm, acc.at[pl.ds(g * _DR, _DR)])

            plsc.subcore_barrier()

            @pl.when(ck < NCK)
            def _():
                def row(i):
                    return msg_hbm.at[ck, pl.ds((sid * GT + i) * _NB, _NB)]

                for b in range(2):
                    pltpu.make_async_copy(row(b), mbufs[b], sems[b]).start()

                def body(jj, _):
                    for b in range(2):
                        i = 2 * jj + b
                        pltpu.make_async_copy(row(i), mbufs[b], sems[b]).wait()
                        pltpu.sync_copy(mbufs[b], acc.at[idx_v.at[i]], add=True)

                        @pl.when(i + 2 < GT)
                        def _():
                            pltpu.make_async_copy(
                                row(i + 2), mbufs[b], sems[b]).start()

                    return 0

                lax.fori_loop(0, GT // 2, body, 0)

            plsc.subcore_barrier()

            @pl.when(ck < NCK)
            def _():
                for jz in range(-(-NZ // 16)):
                    g = jz * 16 + sid

                    @pl.when(g < NZ)
                    def _():
                        pltpu.sync_copy(acc.at[pl.ds(g * _DR, _DR)],
                                        y_hbm.at[ck, pl.ds(g * _DR, _DR)])

            plsc.subcore_barrier()

    return k


@functools.lru_cache(maxsize=None)
def _make_sc_gather(EP, NF, F):
    NG = EP // _NB
    GTW = NG // 32
    mesh = plsc.VectorSubcoreMesh(core_axis_name="c", subcore_axis_name="s")

    @functools.partial(
        pl.kernel,
        out_type=jax.ShapeDtypeStruct((EP, F), jnp.float32),
        mesh=mesh,
        scratch_types=[
            pltpu.VMEM((GTW, _NB), jnp.int32),
            pltpu.VMEM((_NB, F), jnp.float32),
            pltpu.VMEM((_NB, F), jnp.float32),
            pltpu.SemaphoreType.DMA,
            pltpu.SemaphoreType.DMA,
        ],
    )
    def k(tab_hbm, srcr_hbm, xj_hbm, idx_v, rb0, rb1, sg0, sg1):
        w = lax.axis_index("c") * 16 + lax.axis_index("s")
        pltpu.sync_copy(srcr_hbm.at[w], idx_v)
        rbufs, sems = (rb0, rb1), (sg0, sg1)

        def gat(i, b):
            return pltpu.make_async_copy(
                tab_hbm.at[idx_v.at[i]], rbufs[b], sems[b])

        for b in range(2):
            gat(b, b).start()

        def body(jj, _):
            for b in range(2):
                i = 2 * jj + b
                gat(i, b).wait()
                pltpu.sync_copy(
                    rbufs[b], xj_hbm.at[pl.ds((w * GTW + i) * _NB, _NB)])

                @pl.when(i + 2 < GTW)
                def _():
                    gat(i + 2, b).start()

            return 0

        lax.fori_loop(0, GTW // 2, body, 0)

    return k


def _sc_gather(tab, srcr):
    NF, F = tab.shape
    EP = srcr.shape[1] * 32 * _NB
    return _make_sc_gather(EP, NF, F)(tab, srcr)


def _sc_scatter(msg, dstr, N):
    NCK, EPC, _ = msg.shape
    zeros = jnp.zeros((_DR, 128), jnp.float32)
    return _make_sc_scatter(EPC, N, NCK)(msg, dstr, zeros)


def _chunking(C):
    return 3 * C // 16


def _prep_pc(precomp):
    E = precomp.shape[0]
    p = jnp.transpose(precomp, (0, 3, 2, 1)).reshape(E, 2 * _M * _R)
    return jnp.concatenate(
        [p, jnp.zeros((_pad_edges(E) - E, 2 * _M * _R), jnp.float32)])


def _prep_conn(conn):
    E = conn.shape[0]
    return jnp.concatenate(
        [conn, jnp.zeros((_pad_edges(E) - E, 2), jnp.float32)])


def _prep_src(src, Nn):
    E = src.shape[0]
    EP = _pad_edges(E)
    s = jnp.concatenate(
        [src.astype(jnp.int32),
         jnp.full((EP - E,), Nn, jnp.int32)])
    return s.reshape(32, EP // _NB // 32, _NB)


def _prep_dstr(dst):
    E = dst.shape[0]
    EP = _pad_edges(E)
    d = jnp.concatenate(
        [dst.astype(jnp.int32), jnp.zeros((EP - E,), jnp.int32)])
    return jnp.broadcast_to(d.reshape(1, 16, EP // _NB // 16, _NB),
                            (2, 16, EP // _NB // 16, _NB))


def _conv(x, srcp, dstr, pcf, conn, W, b, Nn):
    Mi, C = x.shape[1], x.shape[2]
    NCK = _chunking(C)
    NP = _pad_nodes(Nn)
    F = 2 * Mi * C
    FP = -(-F // 128) * 128
    f = jnp.transpose(x, (0, 3, 1, 2)).reshape(Nn, F)
    f = jnp.concatenate([f, jnp.zeros((Nn, FP - F), jnp.float32)], axis=1)
    f = jnp.concatenate([f, jnp.zeros((1, FP), jnp.float32)])
    xj = _sc_gather(f, srcp)
    msg = _edge_msgs(xj, conn, pcf, Mi, C, NCK)
    y4 = _sc_scatter(msg, dstr, NP)[:, :Nn]
    y3 = jnp.transpose(y4, (1, 0, 2)).reshape(Nn, 2, _R, _M, C)
    y = jnp.transpose(y3, (0, 2, 3, 4, 1))
    out = jnp.einsum('nrmcz,rmco->nmoz', y, W)
    if b is not None:
        out = out.at[:, 0, :, 0].add(b)
    return out


def _c_relu(x, b):
    mag = jnp.sqrt(jnp.sum(x * x, axis=-1, keepdims=True) + 1e-12)
    scale = jax.nn.relu(mag + b[None, None, :, None]) / mag
    return x * scale


def _resnet_block(x, src, dstr, pcf, conn, p, Nn):
    h = _conv(x, src, dstr, pcf, conn, p['W1'], p['b1'], Nn)
    h = _c_relu(h, p['nb1'])
    h = _conv(h, src, dstr, pcf, conn, p['W2'], p['b2'], Nn)
    sc = x
    if sc.shape[1] != h.shape[1]:
        sc = jnp.broadcast_to(sc, (sc.shape[0], h.shape[1], sc.shape[2], sc.shape[3]))
    if 'Ws' in p:
        sc = jnp.einsum('nmcz,co->nmoz', sc, p['Ws'])
    return _c_relu(h + sc, p['nb2'])


def _pool(x_prepool, cluster):
    N = x_prepool.shape[0]
    D = x_prepool.shape[1] * x_prepool.shape[2] * x_prepool.shape[3]
    EP = 2 * _pad_edges(N // 2)
    NCP = _pad_nodes(_N_COARSE)
    pay = jnp.concatenate(
        [x_prepool.reshape(N, D),
         jnp.ones((N, 1), jnp.float32),
         jnp.zeros((N, 128 - D - 1), jnp.float32)], axis=1)
    pay = jnp.concatenate(
        [pay, jnp.zeros((EP - N, 128), jnp.float32)], axis=0)
    msg = pay.reshape(2, EP // 2, 128)
    clp = jnp.concatenate(
        [cluster.astype(jnp.int32), jnp.zeros((EP - N,), jnp.int32)])
    dstr = clp.reshape(2, 16, (EP // 2) // _NB // 16, _NB)
    y = _sc_scatter(msg, dstr, NCP)
    s = y[0, :_N_COARSE] + y[1, :_N_COARSE]
    counts = jnp.maximum(s[:, D], 1.0)
    xp = s[:, :D].reshape(_N_COARSE, _M, 16, 2) / counts[:, None, None, None]
    return xp


def kernel(pos, edge_index0, precomp0, connection0, cluster, edge_index1,
           precomp1, connection1, params):
    src0 = _prep_src(edge_index0[0], _N_NODES)
    src1 = _prep_src(edge_index1[0], _N_COARSE)
    pcf0 = _prep_pc(precomp0)
    pcf1 = _prep_pc(precomp1)
    conn0 = _prep_conn(connection0)
    conn1 = _prep_conn(connection1)
    dstr0 = _prep_dstr(edge_index0[1])
    dstr1 = _prep_dstr(edge_index1[1])

    x = jax.nn.relu(pos @ params['lin0_W'] + params['lin0_b'])
    x = jnp.stack([x, jnp.zeros_like(x)], axis=-1)[:, None, :, :]

    x = _resnet_block(x, src0, dstr0, pcf0, conn0, params['rb11'], _N_NODES)
    x_prepool = _resnet_block(x, src0, dstr0, pcf0, conn0, params['rb12'], _N_NODES)

    xp = _pool(x_prepool, cluster)

    xp = _resnet_block(xp, src1, dstr1, pcf1, conn1, params['rb21'], _N_COARSE)
    xp = _resnet_block(xp, src1, dstr1, pcf1, conn1, params['rb22'], _N_COARSE)
    xp = _resnet_block(xp, src1, dstr1, pcf1, conn1, params['rb31'], _N_COARSE)
    xp = _resnet_block(xp, src1, dstr1, pcf1, conn1, params['rb32'], _N_COARSE)

    x_un = xp[cluster]
    x = jnp.concatenate([x_un, x_prepool], axis=2)

    x = _resnet_block(x, src0, dstr0, pcf0, conn0, params['rb41'], _N_NODES)
    x = _resnet_block(x, src0, dstr0, pcf0, conn0, params['rb42'], _N_NODES)
    x = _conv(x, src0, dstr0, pcf0, conn0, params['convf_W'], None, _N_NODES)

    mag = jnp.sqrt(jnp.sum(x * x, axis=-1) + 1e-12)
    logits = mag.sum(axis=1) + params['bias']
    return jax.nn.log_softmax(logits, axis=1)


# speedup vs baseline: 53.4954x; 1.0058x over previous
import functools

import jax
import jax.numpy as jnp
from jax import lax
from jax.experimental import pallas as pl
from jax.experimental.pallas import tpu as pltpu
from jax.experimental.pallas import tpu_sc as plsc

_N_NODES = 10000
_N_COARSE = 2500
_R = 6
_M = 2
_BE = 1024
_NB = 128
_DR = 400


def _pad_edges(E):
    q = 32 * _NB
    return -(-E // q) * q


def _pad_nodes(N):
    return -(-N // _DR) * _DR


def _edge_body(Mi, C, NCK, xj_ref, conn_ref, pc_ref, msg_ref):
    MiC = Mi * C
    MC = _M * C
    RMC = _R * MC
    xj = xj_ref[...]
    conn = conn_ref[...]
    pc = pc_ref[...]
    cr = conn[:, 0:1]
    ci = conn[:, 1:2]
    rot = []
    for m in range(_M):
        mm = min(m, Mi - 1)
        Xr = xj[:, 0 * MiC + mm * C:0 * MiC + mm * C + C]
        Xi = xj[:, 1 * MiC + mm * C:1 * MiC + mm * C + C]
        if m == 1:
            rot.append((Xr * cr - Xi * ci, Xr * ci + Xi * cr))
        else:
            rot.append((Xr, Xi))
    for z in range(2):
        for r in range(_R):
            for m in range(_M):
                pr = pc[:, 0 * _M * _R + m * _R + r:0 * _M * _R + m * _R + r + 1]
                pi = pc[:, 1 * _M * _R + m * _R + r:1 * _M * _R + m * _R + r + 1]
                Ar, Ai = rot[m]
                val = pr * Ar - pi * Ai if z == 0 else pr * Ai + pi * Ar
                flat = z * RMC + r * MC + m * C
                s = 0
                while s < C:
                    ck, off = (flat + s) // 128, (flat + s) % 128
                    w = min(128 - off, C - s)
                    msg_ref[ck, :, off:off + w] = val[:, s:s + w]
                    s += w


@functools.partial(jax.jit, static_argnames=("Mi", "C", "NCK"))
def _edge_msgs(xj, conn, pcf, Mi, C, NCK):
    E, FP = xj.shape
    grid = E // _BE
    return pl.pallas_call(
        functools.partial(_edge_body, Mi, C, NCK),
        grid=(grid,),
        in_specs=[
            pl.BlockSpec((_BE, FP), lambda i: (i, 0)),
            pl.BlockSpec((_BE, 2), lambda i: (i, 0)),
            pl.BlockSpec((_BE, 2 * _M * _R), lambda i: (i, 0)),
        ],
        out_specs=pl.BlockSpec((NCK, _BE, 128), lambda i: (0, i, 0)),
        out_shape=jax.ShapeDtypeStruct((NCK, E, 128), jnp.float32),
    )(xj, conn, pcf)


@functools.lru_cache(maxsize=None)
def _make_sc_scatter(EPC, N, NCK):
    NG = EPC // _NB
    assert NG % 16 == 0
    GT = NG // 16
    NZ = N // _DR
    mesh = plsc.VectorSubcoreMesh(core_axis_name="c", subcore_axis_name="s")

    @functools.partial(
        pl.kernel,
        out_type=jax.ShapeDtypeStruct((NCK, N, 128), jnp.float32),
        mesh=mesh,
        scratch_types=[
            pltpu.VMEM((GT, _NB), jnp.int32),
            pltpu.VMEM((_NB, 128), jnp.float32),
            pltpu.VMEM((_NB, 128), jnp.float32),
            pltpu.VMEM_SHARED((N, 128), jnp.float32),
            pltpu.SemaphoreType.DMA,
            pltpu.SemaphoreType.DMA,
        ],
    )
    def k(msg_hbm, dstr_hbm, zeros_hbm, y_hbm, idx_v, mb0, mb1, acc, sm0, sm1):
        zc = lax.axis_index("c")
        sid = lax.axis_index("s")
        mbufs, sems = (mb0, mb1), (sm0, sm1)
        pltpu.sync_copy(dstr_hbm.at[zc, sid], idx_v)
        for j in range(-(-NCK // 2)):
            ck = j * 2 + zc

            @pl.when(ck < NCK)
            def _():
                for jz in range(-(-NZ // 16)):
                    g = jz * 16 + sid

                    @pl.when(g < NZ)
                    def _():
                        pltpu.sync_copy(zeros_hbm, acc.at[pl.ds(g * _DR, _DR)])

            plsc.subcore_barrier()

            @pl.when(ck < NCK)
            def _():
                def row(i):
                    return msg_hbm.at[ck, pl.ds((sid * GT + i) * _NB, _NB)]

                for b in range(2):
                    pltpu.make_async_copy(row(b), mbufs[b], sems[b]).start()

                def body(jj, _):
                    for b in range(2):
                        i = 2 * jj + b
                        pltpu.make_async_copy(row(i), mbufs[b], sems[b]).wait()
                        pltpu.sync_copy(mbufs[b], acc.at[idx_v.at[i]], add=True)

                        @pl.when(i + 2 < GT)
                        def _():
                            pltpu.make_async_copy(
                                row(i + 2), mbufs[b], sems[b]).start()

                    return 0

                lax.fori_loop(0, GT // 2, body, 0)

            plsc.subcore_barrier()

            @pl.when(ck < NCK)
            def _():
                for jz in range(-(-NZ // 16)):
                    g = jz * 16 + sid

                    @pl.when(g < NZ)
                    def _():
                        pltpu.sync_copy(acc.at[pl.ds(g * _DR, _DR)],
                                        y_hbm.at[ck, pl.ds(g * _DR, _DR)])

            plsc.subcore_barrier()

    return k


@functools.lru_cache(maxsize=None)
def _make_sc_gather(EP, NF, F):
    NG = EP // _NB
    GTW = NG // 32
    mesh = plsc.VectorSubcoreMesh(core_axis_name="c", subcore_axis_name="s")

    @functools.partial(
        pl.kernel,
        out_type=jax.ShapeDtypeStruct((EP, F), jnp.float32),
        mesh=mesh,
        scratch_types=[
            pltpu.VMEM((GTW, _NB), jnp.int32),
            pltpu.VMEM((_NB, F), jnp.float32),
            pltpu.VMEM((_NB, F), jnp.float32),
            pltpu.SemaphoreType.DMA,
            pltpu.SemaphoreType.DMA,
        ],
    )
    def k(tab_hbm, srcr_hbm, xj_hbm, idx_v, rb0, rb1, sg0, sg1):
        w = lax.axis_index("c") * 16 + lax.axis_index("s")
        pltpu.sync_copy(srcr_hbm.at[w], idx_v)
        rbufs, sems = (rb0, rb1), (sg0, sg1)

        def gat(i, b):
            return pltpu.make_async_copy(
                tab_hbm.at[idx_v.at[i]], rbufs[b], sems[b])

        for b in range(2):
            gat(b, b).start()

        def body(jj, _):
            for b in range(2):
                i = 2 * jj + b
                gat(i, b).wait()
                pltpu.sync_copy(
                    rbufs[b], xj_hbm.at[pl.ds((w * GTW + i) * _NB, _NB)])

                @pl.when(i + 2 < GTW)
                def _():
                    gat(i + 2, b).start()

            return 0

        lax.fori_loop(0, GTW // 2, body, 0)

    return k


def _sc_gather(tab, srcr):
    NF, F = tab.shape
    EP = srcr.shape[1] * 32 * _NB
    return _make_sc_gather(EP, NF, F)(tab, srcr)


def _sc_scatter(msg, dstr, N):
    NCK, EPC, _ = msg.shape
    zeros = jnp.zeros((_DR, 128), jnp.float32)
    return _make_sc_scatter(EPC, N, NCK)(msg, dstr, zeros)


def _chunking(C):
    return 3 * C // 16


def _prep_pc(precomp):
    E = precomp.shape[0]
    p = jnp.transpose(precomp, (0, 3, 2, 1)).reshape(E, 2 * _M * _R)
    return jnp.concatenate(
        [p, jnp.zeros((_pad_edges(E) - E, 2 * _M * _R), jnp.float32)])


def _prep_conn(conn):
    E = conn.shape[0]
    return jnp.concatenate(
        [conn, jnp.zeros((_pad_edges(E) - E, 2), jnp.float32)])


def _prep_src(src, Nn):
    E = src.shape[0]
    EP = _pad_edges(E)
    s = jnp.concatenate(
        [src.astype(jnp.int32),
         jnp.full((EP - E,), Nn, jnp.int32)])
    return s.reshape(32, EP // _NB // 32, _NB)


def _prep_dstr(dst):
    E = dst.shape[0]
    EP = _pad_edges(E)
    d = jnp.concatenate(
        [dst.astype(jnp.int32), jnp.zeros((EP - E,), jnp.int32)])
    return jnp.broadcast_to(d.reshape(1, 16, EP // _NB // 16, _NB),
                            (2, 16, EP // _NB // 16, _NB))


def _conv(x, srcp, dstr, pcf, conn, W, b, Nn):
    Mi, C = x.shape[1], x.shape[2]
    NCK = _chunking(C)
    NP = _pad_nodes(Nn)
    F = 2 * Mi * C
    FP = -(-F // 128) * 128
    f = jnp.transpose(x, (0, 3, 1, 2)).reshape(Nn, F)
    f = jnp.concatenate([f, jnp.zeros((Nn, FP - F), jnp.float32)], axis=1)
    f = jnp.concatenate([f, jnp.zeros((1, FP), jnp.float32)])
    xj = _sc_gather(f, srcp)
    msg = _edge_msgs(xj, conn, pcf, Mi, C, NCK)
    y4 = _sc_scatter(msg, dstr, NP)[:, :Nn]
    y3 = jnp.transpose(y4, (1, 0, 2)).reshape(Nn, 2, _R, _M, C)
    y = jnp.transpose(y3, (0, 2, 3, 4, 1))
    out = jnp.einsum('nrmcz,rmco->nmoz', y, W)
    if b is not None:
        out = out.at[:, 0, :, 0].add(b)
    return out


def _c_relu(x, b):
    mag = jnp.sqrt(jnp.sum(x * x, axis=-1, keepdims=True) + 1e-12)
    scale = jax.nn.relu(mag + b[None, None, :, None]) / mag
    return x * scale


def _resnet_block(x, src, dstr, pcf, conn, p, Nn):
    h = _conv(x, src, dstr, pcf, conn, p['W1'], p['b1'], Nn)
    h = _c_relu(h, p['nb1'])
    h = _conv(h, src, dstr, pcf, conn, p['W2'], p['b2'], Nn)
    sc = x
    if sc.shape[1] != h.shape[1]:
        sc = jnp.broadcast_to(sc, (sc.shape[0], h.shape[1], sc.shape[2], sc.shape[3]))
    if 'Ws' in p:
        sc = jnp.einsum('nmcz,co->nmoz', sc, p['Ws'])
    return _c_relu(h + sc, p['nb2'])


def _pool(x_prepool, cluster):
    N = x_prepool.shape[0]
    D = x_prepool.shape[1] * x_prepool.shape[2] * x_prepool.shape[3]
    EP = 2 * _pad_edges(N // 2)
    NCP = _pad_nodes(_N_COARSE)
    pay = jnp.concatenate(
        [x_prepool.reshape(N, D),
         jnp.ones((N, 1), jnp.float32),
         jnp.zeros((N, 128 - D - 1), jnp.float32)], axis=1)
    pay = jnp.concatenate(
        [pay, jnp.zeros((EP - N, 128), jnp.float32)], axis=0)
    msg = pay.reshape(2, EP // 2, 128)
    clp = jnp.concatenate(
        [cluster.astype(jnp.int32), jnp.zeros((EP - N,), jnp.int32)])
    dstr = clp.reshape(2, 16, (EP // 2) // _NB // 16, _NB)
    y = _sc_scatter(msg, dstr, NCP)
    s = y[0, :_N_COARSE] + y[1, :_N_COARSE]
    counts = jnp.maximum(s[:, D], 1.0)
    xp = s[:, :D].reshape(_N_COARSE, _M, 16, 2) / counts[:, None, None, None]
    return xp


def kernel(pos, edge_index0, precomp0, connection0, cluster, edge_index1,
           precomp1, connection1, params):
    src0 = _prep_src(edge_index0[0], _N_NODES)
    src1 = _prep_src(edge_index1[0], _N_COARSE)
    pcf0 = _prep_pc(precomp0)
    pcf1 = _prep_pc(precomp1)
    conn0 = _prep_conn(connection0)
    conn1 = _prep_conn(connection1)
    dstr0 = _prep_dstr(edge_index0[1])
    dstr1 = _prep_dstr(edge_index1[1])

    x = jax.nn.relu(pos @ params['lin0_W'] + params['lin0_b'])
    x = jnp.stack([x, jnp.zeros_like(x)], axis=-1)[:, None, :, :]

    x = _resnet_block(x, src0, dstr0, pcf0, conn0, params['rb11'], _N_NODES)
    x_prepool = _resnet_block(x, src0, dstr0, pcf0, conn0, params['rb12'], _N_NODES)

    xp = _pool(x_prepool, cluster)

    xp = _resnet_block(xp, src1, dstr1, pcf1, conn1, params['rb21'], _N_COARSE)
    xp = _resnet_block(xp, src1, dstr1, pcf1, conn1, params['rb22'], _N_COARSE)
    xp = _resnet_block(xp, src1, dstr1, pcf1, conn1, params['rb31'], _N_COARSE)
    xp = _resnet_block(xp, src1, dstr1, pcf1, conn1, params['rb32'], _N_COARSE)

    xp_pl = jnp.transpose(xp, (0, 3, 1, 2)).reshape(_N_COARSE, 128)
    EPU = 16384
    clr = jnp.concatenate(
        [cluster.astype(jnp.int32),
         jnp.zeros((EPU - _N_NODES,), jnp.int32)]).reshape(32, EPU // _NB // 32, _NB)
    xun_pl = _sc_gather(xp_pl, clr)[:_N_NODES]
    x_un = jnp.transpose(xun_pl.reshape(_N_NODES, 2, _M, 32), (0, 2, 3, 1))
    x = jnp.concatenate([x_un, x_prepool], axis=2)

    x = _resnet_block(x, src0, dstr0, pcf0, conn0, params['rb41'], _N_NODES)
    x = _resnet_block(x, src0, dstr0, pcf0, conn0, params['rb42'], _N_NODES)
    x = _conv(x, src0, dstr0, pcf0, conn0, params['convf_W'], None, _N_NODES)

    mag = jnp.sqrt(jnp.sum(x * x, axis=-1) + 1e-12)
    logits = mag.sum(axis=1) + params['bias']
    return jax.nn.log_softmax(logits, axis=1)
